# baseline (speedup 1.0000x reference)
import functools

import jax
import jax.numpy as jnp
from jax import lax
from jax.experimental import pallas as pl
from jax.experimental.pallas import tpu as pltpu
from jax.experimental.pallas import tpu_sc as plsc

D = 64
DP = 128
V = 1000000
VP = 1000064
NC = 2
NS = 16
NW = NC * NS
CHUNK = 128
NBUF = 4

BW = 256
NBLK = V // BW
NFULL = 3904
TAIL0 = NBLK * BW


def _mesh():
    return plsc.VectorSubcoreMesh(core_axis_name="c", subcore_axis_name="s")


@jax.jit
def _sc_format(tab_t, tail_t):
    scratch = ([pltpu.VMEM((D, BW + 1), jnp.float32) for _ in range(2)]
               + [pltpu.VMEM((BW, DP), jnp.float32) for _ in range(2)]
               + [pltpu.VMEM((D, D), jnp.float32)]
               + [pltpu.SemaphoreType.DMA((2,)), pltpu.SemaphoreType.DMA((2,))])

    @functools.partial(
        pl.kernel,
        out_type=jax.ShapeDtypeStruct((VP, DP), jnp.float32),
        mesh=_mesh(),
        scratch_types=scratch,
        compiler_params=pltpu.CompilerParams(needs_layout_passes=False),
    )
    def k(tab_hbm, tail_hbm, out_hbm, *rest):
        tile_in = rest[:2]
        rows_out = rest[2:4]
        tail_v = rest[4]
        isem, osem = rest[5], rest[6]
        wid = lax.axis_index("s") * NC + lax.axis_index("c")

        def col0_of(c):
            return c * BW

        def start_in(c, b):
            pltpu.async_copy(tab_hbm.at[:, pl.ds(col0_of(c), BW)],
                             tile_in[b].at[:, pl.ds(0, BW)], isem.at[b])

        def wait_in(c, b):
            pltpu.make_async_copy(tab_hbm.at[:, pl.ds(col0_of(c), BW)],
                                  tile_in[b].at[:, pl.ds(0, BW)],
                                  isem.at[b]).wait()

        def start_out(c, b):
            pltpu.async_copy(rows_out[b], out_hbm.at[pl.ds(col0_of(c), BW)],
                             osem.at[b])

        def wait_out(c, b):
            pltpu.make_async_copy(rows_out[b],
                                  out_hbm.at[pl.ds(col0_of(c), BW)],
                                  osem.at[b]).wait()

        def transpose_ref(src, b, ncl):
            lane = lax.iota(jnp.int32, 16)
            rows = [lane + q * 16 for q in range(4)]

            def load4(cl):
                col = jnp.zeros((16,), jnp.int32) + cl
                return [plsc.load_gather(src, [rows[q], col])
                        for q in range(4)]

            def body(cl, vs):
                nxt = load4(cl + 1)
                for q in range(4):
                    rows_out[b].at[cl][pl.ds(q * 16, 16)] = vs[q]
                return nxt

            vs = lax.fori_loop(0, ncl - 1, body, load4(0), unroll=8)
            for q in range(4):
                rows_out[b].at[ncl - 1][pl.ds(q * 16, 16)] = vs[q]

        def transpose(b):
            transpose_ref(tile_in[b], b, DP)

        nt = NFULL // NW
        extra = NBLK - NFULL

        def first_c(t):
            return wid + t * NW

        start_in(first_c(0), 0)
        start_in(first_c(1), 1)
        for b in range(2):
            wait_in(first_c(b), b)
            transpose(b)
            start_out(first_c(b), b)
            start_in(first_c(b + 2), b)

        def body(p, carry):
            for b in range(2):
                t = 2 * p + b
                wait_in(first_c(t), b)
                wait_out(first_c(t - 2), b)
                transpose(b)
                start_out(first_c(t), b)
                start_in(first_c(t + 2), b)
            return carry

        lax.fori_loop(1, nt // 2 - 1, body, 0)

        for b in range(2):
            t = nt - 2 + b
            wait_in(first_c(t), b)
            wait_out(first_c(t - 2), b)
            transpose(b)
            start_out(first_c(t), b)
        for b in range(2):
            wait_out(first_c(nt - 2 + b), b)

        @pl.when(wid < extra)
        def _():
            c = NFULL + wid
            start_in(c, 0)
            wait_in(c, 0)
            transpose(0)
            start_out(c, 0)
            wait_out(c, 0)

        @pl.when(wid == extra)
        def _():
            pltpu.sync_copy(tail_hbm, tail_v)
            transpose_ref(tail_v, 1, D)
            pltpu.sync_copy(rows_out[1].at[pl.ds(0, D)],
                            out_hbm.at[pl.ds(TAIL0, D)])

    return k(tab_t, tail_t)


@functools.partial(jax.jit, static_argnames=("n",))
def _sc_gather(idx, tabp, n):
    b_per_w = n // NW
    nchunk = b_per_w // CHUNK
    ngroup = nchunk // NBUF

    scratch = ([pltpu.VMEM((CHUNK,), jnp.int32) for _ in range(NBUF)]
               + [pltpu.VMEM((CHUNK, DP), jnp.float32) for _ in range(NBUF)]
               + [pltpu.SemaphoreType.DMA((NBUF,)),
                  pltpu.SemaphoreType.DMA((NBUF,))])

    @functools.partial(
        pl.kernel,
        out_type=jax.ShapeDtypeStruct((n, DP), jnp.float32),
        mesh=_mesh(),
        scratch_types=scratch,
    )
    def k(idx_hbm, table_hbm, out_hbm, *rest):
        idx_v = rest[:NBUF]
        rows_v = rest[NBUF:2 * NBUF]
        gsem, wsem = rest[2 * NBUF], rest[2 * NBUF + 1]
        wid = lax.axis_index("s") * NC + lax.axis_index("c")
        base = wid * b_per_w

        def load_idx(g, b):
            pltpu.sync_copy(idx_hbm.at[pl.ds(base + g * CHUNK, CHUNK)],
                            idx_v[b])

        def start_gather(b):
            pltpu.async_copy(table_hbm.at[idx_v[b]], rows_v[b], gsem.at[b])

        def wait_gather(b):
            pltpu.make_async_copy(table_hbm.at[idx_v[b]], rows_v[b],
                                  gsem.at[b]).wait()

        def start_write(g, b):
            pltpu.async_copy(rows_v[b],
                             out_hbm.at[pl.ds(base + g * CHUNK, CHUNK)],
                             wsem.at[b])

        def wait_write(g, b):
            pltpu.make_async_copy(rows_v[b],
                                  out_hbm.at[pl.ds(base + g * CHUNK, CHUNK)],
                                  wsem.at[b]).wait()

        for b in range(NBUF):
            load_idx(b, b)
            start_gather(b)

        def body(p, carry):
            for b in range(NBUF):
                g = p * NBUF + b
                wait_gather(b)
                start_write(g, b)
                load_idx(g + NBUF, b)
                wait_write(g, b)
                start_gather(b)
            return carry

        lax.fori_loop(0, ngroup - 1, body, 0)

        g0 = (ngroup - 1) * NBUF
        for b in range(NBUF):
            wait_gather(b)
            start_write(g0 + b, b)
        for b in range(NBUF):
            wait_write(g0 + b, b)

    return k(idx, tabp)


def kernel(x, table):
    n = x.shape[0] * x.shape[1]
    idx = x.reshape(-1).astype(jnp.int32)
    tabp = _sc_format(table.T, table[TAIL0:].T)
    out = _sc_gather(idx, tabp, n)
    return out[:, :D].reshape(x.shape + (table.shape[1],))


# speedup vs baseline: 1.4232x; 1.4232x over previous
import functools

import jax
import jax.numpy as jnp
from jax import lax
from jax.experimental import pallas as pl
from jax.experimental.pallas import tpu as pltpu
from jax.experimental.pallas import tpu_sc as plsc

D = 64
DP = 128
V = 1000000
VP = 1000064
NC = 2
NS = 16
NW = NC * NS
CHUNK = 128
NBUF = 4

BW = 256
NBLK = V // BW
NFULL = 3904
TAIL0 = NBLK * BW


def _mesh():
    return plsc.VectorSubcoreMesh(core_axis_name="c", subcore_axis_name="s")


@jax.jit
def _sc_format(tab_t, tail_t):
    scratch = ([pltpu.VMEM((D, BW), jnp.float32) for _ in range(2)]
               + [pltpu.VMEM((BW, DP), jnp.float32) for _ in range(2)]
               + [pltpu.VMEM((D, D), jnp.float32)]
               + [pltpu.SemaphoreType.DMA((2,)), pltpu.SemaphoreType.DMA((2,))])

    @functools.partial(
        pl.kernel,
        out_type=jax.ShapeDtypeStruct((VP, DP), jnp.float32),
        mesh=_mesh(),
        scratch_types=scratch,
        compiler_params=pltpu.CompilerParams(needs_layout_passes=False),
    )
    def k(tab_hbm, tail_hbm, out_hbm, *rest):
        tile_in = rest[:2]
        rows_out = rest[2:4]
        tail_v = rest[4]
        isem, osem = rest[5], rest[6]
        wid = lax.axis_index("s") * NC + lax.axis_index("c")

        def col0_of(c):
            return c * BW

        def start_in(c, b):
            pltpu.async_copy(tab_hbm.at[:, pl.ds(col0_of(c), BW)],
                             tile_in[b], isem.at[b])

        def wait_in(c, b):
            pltpu.make_async_copy(tab_hbm.at[:, pl.ds(col0_of(c), BW)],
                                  tile_in[b], isem.at[b]).wait()

        def start_out(c, b):
            pltpu.async_copy(rows_out[b], out_hbm.at[pl.ds(col0_of(c), BW)],
                             osem.at[b])

        def wait_out(c, b):
            pltpu.make_async_copy(rows_out[b],
                                  out_hbm.at[pl.ds(col0_of(c), BW)],
                                  osem.at[b]).wait()

        def transpose_ref(src, b, ncl):
            lane = lax.iota(jnp.int32, 16)
            rots = [lax.rem(lane + k, 16) for k in range(16)]

            def body(i, carry):
                col = lane + i * 16
                for q in range(4):
                    for k in range(16):
                        row = rots[k] + q * 16
                        v = plsc.load_gather(src, [row, col])
                        plsc.store_scatter(rows_out[b], [col, row], v)
                return carry

            lax.fori_loop(0, ncl // 16, body, 0, unroll=2)

        def transpose(b):
            transpose_ref(tile_in[b], b, DP)

        nt = NFULL // NW
        extra = NBLK - NFULL

        def first_c(t):
            return wid + t * NW

        start_in(first_c(0), 0)
        start_in(first_c(1), 1)
        for b in range(2):
            wait_in(first_c(b), b)
            transpose(b)
            start_out(first_c(b), b)
            start_in(first_c(b + 2), b)

        def body(p, carry):
            for b in range(2):
                t = 2 * p + b
                wait_in(first_c(t), b)
                wait_out(first_c(t - 2), b)
                transpose(b)
                start_out(first_c(t), b)
                start_in(first_c(t + 2), b)
            return carry

        lax.fori_loop(1, nt // 2 - 1, body, 0)

        for b in range(2):
            t = nt - 2 + b
            wait_in(first_c(t), b)
            wait_out(first_c(t - 2), b)
            transpose(b)
            start_out(first_c(t), b)
        for b in range(2):
            wait_out(first_c(nt - 2 + b), b)

        @pl.when(wid < extra)
        def _():
            c = NFULL + wid
            start_in(c, 0)
            wait_in(c, 0)
            transpose(0)
            start_out(c, 0)
            wait_out(c, 0)

        @pl.when(wid == extra)
        def _():
            pltpu.sync_copy(tail_hbm, tail_v)
            transpose_ref(tail_v, 1, D)
            pltpu.sync_copy(rows_out[1].at[pl.ds(0, D)],
                            out_hbm.at[pl.ds(TAIL0, D)])

    return k(tab_t, tail_t)


@functools.partial(jax.jit, static_argnames=("n",))
def _sc_gather(idx, tabp, n):
    b_per_w = n // NW
    nchunk = b_per_w // CHUNK
    ngroup = nchunk // NBUF

    scratch = ([pltpu.VMEM((CHUNK,), jnp.int32) for _ in range(NBUF)]
               + [pltpu.VMEM((CHUNK, DP), jnp.float32) for _ in range(NBUF)]
               + [pltpu.SemaphoreType.DMA((NBUF,)),
                  pltpu.SemaphoreType.DMA((NBUF,))])

    @functools.partial(
        pl.kernel,
        out_type=jax.ShapeDtypeStruct((n, DP), jnp.float32),
        mesh=_mesh(),
        scratch_types=scratch,
    )
    def k(idx_hbm, table_hbm, out_hbm, *rest):
        idx_v = rest[:NBUF]
        rows_v = rest[NBUF:2 * NBUF]
        gsem, wsem = rest[2 * NBUF], rest[2 * NBUF + 1]
        wid = lax.axis_index("s") * NC + lax.axis_index("c")
        base = wid * b_per_w

        def load_idx(g, b):
            pltpu.sync_copy(idx_hbm.at[pl.ds(base + g * CHUNK, CHUNK)],
                            idx_v[b])

        def start_gather(b):
            pltpu.async_copy(table_hbm.at[idx_v[b]], rows_v[b], gsem.at[b])

        def wait_gather(b):
            pltpu.make_async_copy(table_hbm.at[idx_v[b]], rows_v[b],
                                  gsem.at[b]).wait()

        def start_write(g, b):
            pltpu.async_copy(rows_v[b],
                             out_hbm.at[pl.ds(base + g * CHUNK, CHUNK)],
                             wsem.at[b])

        def wait_write(g, b):
            pltpu.make_async_copy(rows_v[b],
                                  out_hbm.at[pl.ds(base + g * CHUNK, CHUNK)],
                                  wsem.at[b]).wait()

        for b in range(NBUF):
            load_idx(b, b)
            start_gather(b)

        def body(p, carry):
            for b in range(NBUF):
                g = p * NBUF + b
                wait_gather(b)
                start_write(g, b)
                load_idx(g + NBUF, b)
                wait_write(g, b)
                start_gather(b)
            return carry

        lax.fori_loop(0, ngroup - 1, body, 0)

        g0 = (ngroup - 1) * NBUF
        for b in range(NBUF):
            wait_gather(b)
            start_write(g0 + b, b)
        for b in range(NBUF):
            wait_write(g0 + b, b)

    return k(idx, tabp)


def kernel(x, table):
    n = x.shape[0] * x.shape[1]
    idx = x.reshape(-1).astype(jnp.int32)
    tabp = _sc_format(table.T, table[TAIL0:].T)
    out = _sc_gather(idx, tabp, n)
    return out[:, :D].reshape(x.shape + (table.shape[1],))
